# Initial kernel scaffold; baseline (speedup 1.0000x reference)
import functools

import jax
import jax.numpy as jnp
from jax import lax
from jax.experimental import pallas as pl
from jax.experimental.pallas import tpu as pltpu
from jax.experimental.pallas import tpu_sc as plsc

NN = 10000
EE = 640000
NPAD = 10240
NW = 32
EPT = 20480
EP = NW * EPT
CHUNK = 2048
NCH = EPT // CHUNK
GSUB = 128
NEG = 0.01


def _sc_mesh():
    return plsc.VectorSubcoreMesh(core_axis_name="c", subcore_axis_name="s")


def _gather_eg(src_p, dst_p, edge_embeds):
    P = edge_embeds.shape[0]

    @functools.partial(
        pl.kernel,
        out_type=jax.ShapeDtypeStruct((3, EP), jnp.float32),
        mesh=_sc_mesh(),
        scratch_types=[
            pltpu.VMEM((CHUNK,), jnp.int32),
            pltpu.VMEM((CHUNK,), jnp.int32),
            pltpu.VMEM((CHUNK,), jnp.int32),
            pltpu.VMEM((CHUNK, 3), jnp.float32),
            pltpu.VMEM((3, CHUNK), jnp.float32),
            pltpu.SemaphoreType.DMA,
        ],
    )
    def k(src_hbm, dst_hbm, ee_hbm, eg_hbm, sbuf, dbuf, ibuf, rowbuf, colbuf, sem):
        wid = lax.axis_index("c") * 16 + lax.axis_index("s")
        base = wid * EPT
        iota = lax.iota(jnp.int32, 16)

        def chunk_body(ch, carry):
            e0 = base + ch * CHUNK
            pltpu.sync_copy(src_hbm.at[pl.ds(e0, CHUNK)], sbuf)
            pltpu.sync_copy(dst_hbm.at[pl.ds(e0, CHUNK)], dbuf)

            def idx_body(kk, c2):
                s = sbuf[pl.ds(kk * 16, 16)]
                d = dbuf[pl.ds(kk * 16, 16)]
                a = jnp.minimum(s, d)
                b = jnp.maximum(s, d)
                v = b - 1 + a * NN - lax.shift_right_arithmetic(a * (a + 3), 1)
                v = jnp.where(v < 0, v + P, v)
                ibuf[pl.ds(kk * 16, 16)] = v
                return c2

            lax.fori_loop(0, CHUNK // 16, idx_body, 0, unroll=2)

            copies = []
            for j in range(CHUNK // GSUB):
                copies.append(
                    pltpu.async_copy(
                        ee_hbm.at[ibuf.at[pl.ds(j * GSUB, GSUB)]],
                        rowbuf.at[pl.ds(j * GSUB, GSUB)],
                        sem,
                    )
                )
            for cpy in copies:
                cpy.wait()

            for c in range(3):
                cvec = jnp.full((16,), c, jnp.int32)

                def t_body(kk, c2, cvec=cvec, c=c):
                    rowi = iota + kk * 16
                    col = plsc.load_gather(rowbuf, [rowi, cvec])
                    colbuf[c, pl.ds(kk * 16, 16)] = col
                    return c2

                lax.fori_loop(0, CHUNK // 16, t_body, 0, unroll=2)
            for c in range(3):
                pltpu.sync_copy(colbuf.at[c], eg_hbm.at[c, pl.ds(e0, CHUNK)])
            return carry

        lax.fori_loop(0, NCH, chunk_body, 0)

    return k(src_p, dst_p, edge_embeds)


def _msg_pass(xT, src_p, dst_p, eg):
    @functools.partial(
        pl.kernel,
        out_type=jax.ShapeDtypeStruct((NW, 3, NPAD), jnp.float32),
        mesh=_sc_mesh(),
        scratch_types=[
            pltpu.VMEM((3, NPAD), jnp.float32),
            pltpu.VMEM((3, NPAD), jnp.float32),
            pltpu.VMEM((CHUNK,), jnp.int32),
            pltpu.VMEM((CHUNK,), jnp.int32),
            pltpu.VMEM((3, CHUNK), jnp.float32),
        ],
    )
    def k(x_hbm, src_hbm, dst_hbm, eg_hbm, out_hbm, xbuf, acc, sbuf, dbuf, egbuf):
        wid = lax.axis_index("c") * 16 + lax.axis_index("s")
        base = wid * EPT
        pltpu.sync_copy(x_hbm, xbuf)

        zero = jnp.zeros((16,), jnp.float32)
        for c in range(3):

            def zb(i, c2, c=c):
                acc[c, pl.ds(i * 16, 16)] = zero
                return c2

            lax.fori_loop(0, NPAD // 16, zb, 0, unroll=4)

        cvecs = [jnp.full((16,), c, jnp.int32) for c in range(3)]

        def chunk_body(ch, carry):
            e0 = base + ch * CHUNK
            pltpu.sync_copy(src_hbm.at[pl.ds(e0, CHUNK)], sbuf)
            pltpu.sync_copy(dst_hbm.at[pl.ds(e0, CHUNK)], dbuf)
            for c in range(3):
                pltpu.sync_copy(eg_hbm.at[c, pl.ds(e0, CHUNK)], egbuf.at[c])

            def e_body(kk, c2):
                s = sbuf[pl.ds(kk * 16, 16)]
                d = dbuf[pl.ds(kk * 16, 16)]
                for c in range(3):
                    xc = plsc.load_gather(xbuf, [cvecs[c], s])
                    egc = egbuf[c, pl.ds(kk * 16, 16)]
                    msg = jnp.maximum(xc + egc, 0.0)
                    plsc.addupdate_scatter(acc, [cvecs[c], d], msg)
                return c2

            lax.fori_loop(0, CHUNK // 16, e_body, 0, unroll=2)
            return carry

        lax.fori_loop(0, NCH, chunk_body, 0)
        pltpu.sync_copy(acc, out_hbm.at[wid])

    return k(xT, src_p, dst_p, eg)


def _mlp(partials, xT, p):
    pr = partials.reshape(NW * 3, NPAD)
    BN = 1280
    grid = NPAD // BN

    def body(pp, xx, W1, b1, W2, b2, W3, b3, W4, b4, out):
        pv = pp[...]
        neigh = jnp.sum(pv.reshape(NW, 3, BN), axis=0)
        x = xx[...]
        h = jnp.concatenate([neigh, x], axis=0)

        def dense(h, Wref, bref, nin, nout):
            outs = []
            for o in range(nout):
                acc = h[0:1, :] * Wref[0, o] + bref[o]
                for i in range(1, nin):
                    acc = acc + h[i:i + 1, :] * Wref[i, o]
                outs.append(acc)
            return jnp.concatenate(outs, axis=0)

        def lrelu(v):
            return jnp.maximum(v, v * NEG)

        h = lrelu(dense(h, W1, b1, 6, 6))
        h = lrelu(dense(h, W2, b2, 6, 6))
        h = lrelu(dense(h, W3, b3, 6, 3))
        out[...] = dense(h, W4, b4, 3, 3)

    wspec = pl.BlockSpec(memory_space=pltpu.SMEM)
    return pl.pallas_call(
        body,
        grid=(grid,),
        in_specs=[
            pl.BlockSpec((NW * 3, BN), lambda i: (0, i)),
            pl.BlockSpec((3, BN), lambda i: (0, i)),
            wspec, wspec, wspec, wspec, wspec, wspec, wspec, wspec,
        ],
        out_specs=pl.BlockSpec((3, BN), lambda i: (0, i)),
        out_shape=jax.ShapeDtypeStruct((3, NPAD), jnp.float32),
    )(pr, xT, p["W1"], p["b1"], p["W2"], p["b2"],
      p["W3"], p["b3"], p["W4"], p["b4"])


def kernel(node_embeds, edge_embeds, edge_index, params):
    src = edge_index[:, 0].astype(jnp.int32)
    dst = edge_index[:, 1].astype(jnp.int32)
    nextra = EP - EE
    src_p = jnp.concatenate([src, jnp.arange(nextra, dtype=jnp.int32) % 8192])
    dst_p = jnp.concatenate([dst, jnp.full((nextra,), NN, jnp.int32)])
    xT = jnp.zeros((3, NPAD), jnp.float32).at[:, :NN].set(node_embeds.T)

    eg = _gather_eg(src_p, dst_p, edge_embeds)
    x = xT
    for p in params:
        partials = _msg_pass(x, src_p, dst_p, eg)
        x = _mlp(partials, x, p)
    return x[:, :NN].T


# speedup vs baseline: 3.0727x; 3.0727x over previous
import functools

import jax
import jax.numpy as jnp
from jax import lax
from jax.experimental import pallas as pl
from jax.experimental.pallas import tpu as pltpu
from jax.experimental.pallas import tpu_sc as plsc

NN = 10000
EE = 640000
NPAD = 10240
NW = 32
EPT = 20480
EP = NW * EPT
CHUNK = 2048
NCH = EPT // CHUNK
GSUB = 128
NEG = 0.01


def _sc_mesh():
    return plsc.VectorSubcoreMesh(core_axis_name="c", subcore_axis_name="s")


def _gather_eg(src_p, dst_p, ee0, ee1, ee2):
    P = ee0.shape[0]

    @functools.partial(
        pl.kernel,
        out_type=jax.ShapeDtypeStruct((3 * EP,), jnp.float32),
        mesh=_sc_mesh(),
        compiler_params=pltpu.CompilerParams(needs_layout_passes=False),
        scratch_types=[
            pltpu.VMEM((CHUNK,), jnp.int32),
            pltpu.VMEM((CHUNK,), jnp.int32),
            pltpu.VMEM((CHUNK,), jnp.int32),
            pltpu.VMEM((3 * CHUNK,), jnp.float32),
            pltpu.SemaphoreType.DMA,
        ],
    )
    def k(src_hbm, dst_hbm, e0_hbm, e1_hbm, e2_hbm, eg_hbm,
          sbuf, dbuf, ibuf, colbuf, sem):
        wid = lax.axis_index("c") * 16 + lax.axis_index("s")
        base = wid * EPT
        planes = (e0_hbm, e1_hbm, e2_hbm)

        def chunk_body(ch, carry):
            e0 = base + ch * CHUNK
            pltpu.sync_copy(src_hbm.at[pl.ds(e0, CHUNK)], sbuf)
            pltpu.sync_copy(dst_hbm.at[pl.ds(e0, CHUNK)], dbuf)

            def idx_body(kk, c2):
                s = sbuf[pl.ds(kk * 16, 16)]
                d = dbuf[pl.ds(kk * 16, 16)]
                a = jnp.minimum(s, d)
                b = jnp.maximum(s, d)
                v = b - 1 + a * NN - lax.shift_right_arithmetic(a * (a + 3), 1)
                v = jnp.where(v < 0, v + P, v)
                ibuf[pl.ds(kk * 16, 16)] = v
                return c2

            lax.fori_loop(0, CHUNK // 16, idx_body, 0, unroll=2)

            copies = []
            for c in range(3):
                for j in range(CHUNK // GSUB):
                    copies.append(
                        pltpu.async_copy(
                            planes[c].at[ibuf.at[pl.ds(j * GSUB, GSUB)]],
                            colbuf.at[pl.ds(c * CHUNK + j * GSUB, GSUB)],
                            sem,
                        )
                    )
            for cpy in copies:
                cpy.wait()
            for c in range(3):
                pltpu.sync_copy(colbuf.at[pl.ds(c * CHUNK, CHUNK)],
                                eg_hbm.at[pl.ds(c * EP + e0, CHUNK)])
            return carry

        lax.fori_loop(0, NCH, chunk_body, 0)

    return k(src_p, dst_p, ee0, ee1, ee2)


def _msg_pass(xflat, src_p, dst_p, egflat):
    @functools.partial(
        pl.kernel,
        out_type=jax.ShapeDtypeStruct((NW, 3 * NPAD), jnp.float32),
        mesh=_sc_mesh(),
        compiler_params=pltpu.CompilerParams(needs_layout_passes=False),
        scratch_types=[
            pltpu.VMEM((3 * NPAD,), jnp.float32),
            pltpu.VMEM((3 * NPAD,), jnp.float32),
            pltpu.VMEM((CHUNK,), jnp.int32),
            pltpu.VMEM((CHUNK,), jnp.int32),
            pltpu.VMEM((3 * CHUNK,), jnp.float32),
        ],
    )
    def k(x_hbm, src_hbm, dst_hbm, eg_hbm, out_hbm, xbuf, acc, sbuf, dbuf, egbuf):
        wid = lax.axis_index("c") * 16 + lax.axis_index("s")
        base = wid * EPT
        pltpu.sync_copy(x_hbm, xbuf)

        zero = jnp.zeros((16,), jnp.float32)

        def zb(i, c2):
            acc[pl.ds(i * 16, 16)] = zero
            return c2

        lax.fori_loop(0, 3 * NPAD // 16, zb, 0, unroll=8)

        def chunk_body(ch, carry):
            e0 = base + ch * CHUNK
            pltpu.sync_copy(src_hbm.at[pl.ds(e0, CHUNK)], sbuf)
            pltpu.sync_copy(dst_hbm.at[pl.ds(e0, CHUNK)], dbuf)
            for c in range(3):
                pltpu.sync_copy(eg_hbm.at[pl.ds(c * EP + e0, CHUNK)],
                                egbuf.at[pl.ds(c * CHUNK, CHUNK)])

            def e_body(kk, c2):
                s = sbuf[pl.ds(kk * 16, 16)]
                d = dbuf[pl.ds(kk * 16, 16)]
                for c in range(3):
                    xc = plsc.load_gather(xbuf, [s + (c * NPAD)])
                    egc = egbuf[pl.ds(c * CHUNK + kk * 16, 16)]
                    msg = jnp.maximum(xc + egc, 0.0)
                    plsc.addupdate_scatter(acc, [d + (c * NPAD)], msg)
                return c2

            lax.fori_loop(0, CHUNK // 16, e_body, 0, unroll=2)
            return carry

        lax.fori_loop(0, NCH, chunk_body, 0)
        pltpu.sync_copy(acc, out_hbm.at[wid])

    return k(xflat, src_p, dst_p, egflat)


def _mlp(partials, xT, p):
    pr = partials.reshape(NW * 3, NPAD)
    BN = 1280
    grid = NPAD // BN

    def body(pp, xx, W1, b1, W2, b2, W3, b3, W4, b4, out):
        pv = pp[...]
        neigh = jnp.sum(pv.reshape(NW, 3, BN), axis=0)
        x = xx[...]
        h = jnp.concatenate([neigh, x], axis=0)

        def dense(h, Wref, bref, nin, nout):
            outs = []
            for o in range(nout):
                acc = h[0:1, :] * Wref[0, o] + bref[o]
                for i in range(1, nin):
                    acc = acc + h[i:i + 1, :] * Wref[i, o]
                outs.append(acc)
            return jnp.concatenate(outs, axis=0)

        def lrelu(v):
            return jnp.maximum(v, v * NEG)

        h = lrelu(dense(h, W1, b1, 6, 6))
        h = lrelu(dense(h, W2, b2, 6, 6))
        h = lrelu(dense(h, W3, b3, 6, 3))
        out[...] = dense(h, W4, b4, 3, 3)

    wspec = pl.BlockSpec(memory_space=pltpu.SMEM)
    return pl.pallas_call(
        body,
        grid=(grid,),
        in_specs=[
            pl.BlockSpec((NW * 3, BN), lambda i: (0, i)),
            pl.BlockSpec((3, BN), lambda i: (0, i)),
            wspec, wspec, wspec, wspec, wspec, wspec, wspec, wspec,
        ],
        out_specs=pl.BlockSpec((3, BN), lambda i: (0, i)),
        out_shape=jax.ShapeDtypeStruct((3, NPAD), jnp.float32),
    )(pr, xT, p["W1"], p["b1"], p["W2"], p["b2"],
      p["W3"], p["b3"], p["W4"], p["b4"])


def kernel(node_embeds, edge_embeds, edge_index, params):
    src = edge_index[:, 0].astype(jnp.int32)
    dst = edge_index[:, 1].astype(jnp.int32)
    nextra = EP - EE
    src_p = jnp.concatenate([src, jnp.arange(nextra, dtype=jnp.int32) % 8192])
    dst_p = jnp.concatenate([dst, jnp.full((nextra,), NN, jnp.int32)])
    xT = jnp.zeros((3, NPAD), jnp.float32).at[:, :NN].set(node_embeds.T)

    ee0 = edge_embeds[:, 0]
    ee1 = edge_embeds[:, 1]
    ee2 = edge_embeds[:, 2]

    eg = _gather_eg(src_p, dst_p, ee0, ee1, ee2)
    x = xT
    for p in params:
        partials = _msg_pass(x.reshape(-1), src_p, dst_p, eg)
        x = _mlp(partials, x, p)
    return x[:, :NN].T
